# Initial kernel scaffold; baseline (speedup 1.0000x reference)
import jax
import jax.numpy as jnp
from jax.experimental import pallas as pl


def kernel(x, edge_index, edge_attr, masked_atom_mask, atom_emb1, atom_emb2, bond_emb1, bond_emb2, W1, b1, W2, b2, bn_scale, bn_bias, prelu_a, enc2dec_W, dec_bond_emb1, dec_bond_emb2, dec_W1, dec_b1, dec_W2, dec_b2):
    raise NotImplementedError("write your pallas kernel here")


# speedup vs baseline: 1.7513x; 1.7513x over previous
import functools

import jax
import jax.numpy as jnp
from jax import lax
from jax.experimental import pallas as pl
from jax.experimental.pallas import tpu as pltpu
from jax.experimental.pallas import tpu_sc as plsc

N = 10000
NP = 10240
E = 160000
EP = 163840
F = 300
FQ = 80
FP = 4 * FQ
HID = 600
ODIM = 119
OP = 128
NL = 5
CH = 128
B = 1000
NBLK = N // B

_mesh = plsc.VectorSubcoreMesh(core_axis_name="c", subcore_axis_name="s",
                               num_cores=2, num_subcores=16)
_f32 = jnp.float32
_i32 = jnp.int32
_SC_PARAMS = pltpu.CompilerParams(needs_layout_passes=False,
                                  use_tc_tiling_on_sc=False)


def _split_cols(a):
    z = jnp.zeros(a.shape[:-1] + (5,), a.dtype)
    parts = []
    for q in range(4):
        parts += [a[..., 75 * q:75 * (q + 1)], z]
    return jnp.concatenate(parts, axis=-1)


def _split_rows(a):
    z = jnp.zeros((5,) + a.shape[1:], a.dtype)
    parts = []
    for q in range(4):
        parts += [a[75 * q:75 * (q + 1)], z]
    return jnp.concatenate(parts, axis=0)


def _zero_rows_loop(ref, nrows, ncolgroups):
    zero16 = jnp.zeros((16,), _f32)

    def body(i, _):
        for j in range(ncolgroups):
            ref[i, pl.ds(j * 16, 16)] = zero16
        return 0

    lax.fori_loop(0, nrows, body, 0)


@functools.partial(
    pl.kernel,
    out_type=[jax.ShapeDtypeStruct((4 * NP, FQ), _f32),
              jax.ShapeDtypeStruct((2 * NP, 16), _f32)],
    mesh=_mesh,
    scratch_types=[
        pltpu.VMEM((640,), _i32),
        pltpu.VMEM((CH, FQ), _f32),
        pltpu.VMEM((CH, 16), _f32),
        pltpu.VMEM((40, CH), _i32),
        pltpu.VMEM((40, CH), _i32),
        pltpu.VMEM_SHARED((NP, 16), _f32),
        pltpu.SemaphoreType.DMA,
    ],
    compiler_params=_SC_PARAMS,
)
def _sc_init_kernel(atab_hbm, ax4_hbm, cvals_hbm, dst2_hbm,
                    h0_out, cnt_out, axv, rows, cbuf, cv, dstv, cnt_sh, sem):
    c = lax.axis_index("c")
    s = lax.axis_index("s")

    for j in range(2):
        q = 2 * c + j
        pltpu.sync_copy(ax4_hbm.at[pl.ds(q * NP + s * 640, 640)], axv)
        for k in range(5):
            pltpu.async_copy(atab_hbm.at[axv.at[pl.ds(k * CH, CH)]], rows,
                             sem).wait()
            pltpu.sync_copy(rows,
                            h0_out.at[pl.ds(q * NP + s * 640 + k * CH, CH)])

    _zero_rows_loop(cbuf, CH, 1)
    for k in range(5):
        pltpu.sync_copy(cbuf, cnt_sh.at[pl.ds(s * 640 + k * CH, CH)])
    pltpu.sync_copy(cvals_hbm.at[pl.ds(c * 640 + s * 40, 40)], cv)
    pltpu.sync_copy(dst2_hbm.at[pl.ds(c * 640 + s * 40, 40)], dstv)
    plsc.subcore_barrier()

    ones16 = jnp.ones((16,), _f32)
    zero16 = jnp.zeros((16,), _f32)
    iota16 = lax.iota(_i32, 16)

    def chunk(k, _):
        for g in range(8):
            rowi = iota16 + g * 16
            coli = cv[k, pl.ds(g * 16, 16)]
            plsc.store_scatter(cbuf, [rowi, coli], ones16)
        pltpu.sync_copy(cbuf, cnt_sh.at[dstv.at[k]], add=True)
        for g in range(8):
            rowi = iota16 + g * 16
            coli = cv[k, pl.ds(g * 16, 16)]
            plsc.store_scatter(cbuf, [rowi, coli], zero16)
        return 0

    lax.fori_loop(0, 40, chunk, 0)
    plsc.subcore_barrier()
    pltpu.sync_copy(cnt_sh.at[pl.ds(s * 640, 640)],
                    cnt_out.at[pl.ds(c * NP + s * 640, 640)])


def _make_sc_agg(h_rows):
    @functools.partial(
        pl.kernel,
        out_type=jax.ShapeDtypeStruct((4 * NP, FQ), _f32),
        mesh=_mesh,
        scratch_types=[
            pltpu.VMEM((80, CH), _i32),
            pltpu.VMEM((80, CH), _i32),
            pltpu.VMEM((80, CH), _i32),
            pltpu.VMEM((CH, FQ), _f32),
            pltpu.VMEM((CH, FQ), _f32),
            pltpu.VMEM((CH, FQ), _f32),
            pltpu.VMEM_SHARED((NP, FQ), _f32),
            pltpu.SemaphoreType.DMA,
            pltpu.SemaphoreType.DMA,
        ],
        compiler_params=_SC_PARAMS,
    )
    def k(h_hbm, bt_hbm, src4_hbm, c4_hbm, dst2_hbm, out_hbm,
          srcv, dstv, cv, rows, brows, zrows, acc, sem, sem2):
        c = lax.axis_index("c")
        s = lax.axis_index("s")
        pltpu.sync_copy(dst2_hbm.at[pl.ds(s * 80, 80)], dstv)
        _zero_rows_loop(zrows, CH, FQ // 16)
        for j in range(2):
            q = 2 * c + j
            pltpu.sync_copy(src4_hbm.at[pl.ds(q * 1280 + s * 80, 80)], srcv)
            pltpu.sync_copy(c4_hbm.at[pl.ds(q * 1280 + s * 80, 80)], cv)
            for k0 in range(5):
                pltpu.sync_copy(zrows, acc.at[pl.ds(s * 640 + k0 * CH, CH)])
            plsc.subcore_barrier()

            def chunk(k1, _):
                cp1 = pltpu.async_copy(h_hbm.at[srcv.at[k1]], rows, sem)
                cp2 = pltpu.async_copy(bt_hbm.at[cv.at[k1]], brows, sem2)
                cp1.wait()
                cp2.wait()

                def addrow(i, _):
                    for g in range(FQ // 16):
                        sl = pl.ds(g * 16, 16)
                        rows[i, sl] = rows[i, sl] + brows[i, sl]
                    return 0

                lax.fori_loop(0, CH, addrow, 0)
                pltpu.sync_copy(rows, acc.at[dstv.at[k1]], add=True)
                return 0

            lax.fori_loop(0, 80, chunk, 0)
            plsc.subcore_barrier()
            pltpu.sync_copy(acc.at[pl.ds(s * 640, 640)],
                            out_hbm.at[pl.ds(q * NP + s * 640, 640)])

    def run(h_flat, btq, src4, c4, dst2):
        assert h_flat.shape == (h_rows, FQ)
        return k(h_flat, btq, src4, c4, dst2)

    return run


def _compact(ref4):
    return jnp.concatenate([ref4[q][:, :75] for q in range(4)], axis=-1)


def _to_quarters(hh, out_ref, bsz):
    z5 = jnp.zeros((bsz, 5), _f32)
    for q in range(4):
        out_ref[q] = jnp.concatenate([hh[:, 75 * q:75 * (q + 1)], z5], axis=-1)


def _assemble(agg_ref, h_ref, selfr_ref):
    return _compact(agg_ref) + (_compact(h_ref) + selfr_ref[...])


def _layer_body(agg_ref, h_ref, selfr_ref, w1_ref, b1_ref,
                w2_ref, b2_ref, sc_ref, bi_ref, pa_ref, keep_ref, e2d_ref,
                out_ref, y_scr, st_scr, *, mode):
    p = pl.program_id(0)
    i = pl.program_id(1)

    @pl.when(p == 0)
    def _pass0():
        a = _assemble(agg_ref, h_ref, selfr_ref)
        hmid = jnp.maximum(
            jnp.dot(a, w1_ref[...], preferred_element_type=_f32)
            + b1_ref[...], 0.0)
        y = jnp.dot(hmid, w2_ref[...], preferred_element_type=_f32) + b2_ref[...]
        y_scr[pl.ds(i * B, B), :] = y

        @pl.when(i == 0)
        def _init():
            st_scr[...] = jnp.zeros_like(st_scr)

        st_scr[0:1, :] += jnp.sum(y, axis=0, keepdims=True)

    @pl.when(p == 1)
    def _pass1():
        y = y_scr[pl.ds(i * B, B), :]
        mu = st_scr[0:1, :] / float(N)
        d = y - mu
        st_scr[1:2, :] += jnp.sum(d * d, axis=0, keepdims=True)

    @pl.when(p == 2)
    def _pass2():
        y = y_scr[pl.ds(i * B, B), :]
        mu = st_scr[0:1, :] / float(N)
        var = st_scr[1:2, :] / float(N)
        hh = (y - mu) / jnp.sqrt(var + 1e-5) * sc_ref[...] + bi_ref[...]
        if mode == "relu":
            hh = jnp.maximum(hh, 0.0)
            _to_quarters(hh, out_ref, B)
        else:
            hh = jnp.where(hh >= 0.0, hh, pa_ref[0, 0] * hh)
            hd = jnp.dot(hh, e2d_ref[...], preferred_element_type=_f32)
            hd = hd * keep_ref[...]
            _to_quarters(hd, out_ref, B)


def _dec_body(agg_ref, h_ref, selfr_ref, w1_ref, b1_ref,
              w2_ref, b2_ref, out_ref):
    a = _assemble(agg_ref, h_ref, selfr_ref)
    hmid = jnp.maximum(
        jnp.dot(a, w1_ref[...], preferred_element_type=_f32) + b1_ref[...],
        0.0)
    out_ref[...] = (jnp.dot(hmid, w2_ref[...], preferred_element_type=_f32)
                    + b2_ref[...])


def _p0_spec(bs):
    return pl.BlockSpec(bs, lambda p, i: (0, i * (1 - p) * (2 - p) // 2, 0))


def _full2(shape):
    return pl.BlockSpec(shape, lambda p, i: (0, 0))


_CPARAMS = pltpu.CompilerParams(
    dimension_semantics=("arbitrary", "arbitrary"),
    vmem_limit_bytes=110 * 1024 * 1024,
)


def _layer_call(aggsc, h, selfr, w1, b1, w2, b2, scp, bip, pa, keep,
                e2d, mode):
    body = functools.partial(_layer_body, mode=mode)
    lastp = lambda p, i: (i * (p // 2), 0)
    return pl.pallas_call(
        body,
        grid=(3, NBLK),
        in_specs=[
            _p0_spec((4, B, FQ)),
            _p0_spec((4, B, FQ)),
            _full2((1, F)),
            _full2((F, HID)),
            _full2((1, HID)),
            _full2((HID, F)),
            _full2((1, F)),
            _full2((1, F)),
            _full2((1, F)),
            _full2((1, 1)),
            pl.BlockSpec((B, 1), lastp),
            _full2((F, F)),
        ],
        out_specs=pl.BlockSpec((4, B, FQ), lambda p, i: (0, i * (p // 2), 0)),
        out_shape=jax.ShapeDtypeStruct((4, N, FQ), _f32),
        scratch_shapes=[pltpu.VMEM((N, F), _f32),
                        pltpu.VMEM((8, F), _f32)],
        compiler_params=_CPARAMS,
    )(aggsc, h, selfr, w1, b1, w2, b2, scp, bip, pa, keep, e2d)


def _dec_call(aggsc, h, selfr, w1, b1, w2, b2):
    return pl.pallas_call(
        _dec_body,
        grid=(NBLK,),
        in_specs=[
            pl.BlockSpec((4, B, FQ), lambda i: (0, i, 0)),
            pl.BlockSpec((4, B, FQ), lambda i: (0, i, 0)),
            pl.BlockSpec((1, F), lambda i: (0, 0)),
            pl.BlockSpec((F, HID), lambda i: (0, 0)),
            pl.BlockSpec((1, HID), lambda i: (0, 0)),
            pl.BlockSpec((HID, OP), lambda i: (0, 0)),
            pl.BlockSpec((1, OP), lambda i: (0, 0)),
        ],
        out_specs=pl.BlockSpec((B, OP), lambda i: (i, 0)),
        out_shape=jax.ShapeDtypeStruct((N, OP), _f32),
        compiler_params=pltpu.CompilerParams(
            dimension_semantics=("arbitrary",),
            vmem_limit_bytes=110 * 1024 * 1024,
        ),
    )(aggsc, h, selfr, w1, b1, w2, b2)


def kernel(x, edge_index, edge_attr, masked_atom_mask, atom_emb1, atom_emb2,
           bond_emb1, bond_emb2, W1, b1, W2, b2, bn_scale, bn_bias,
           prelu_a, enc2dec_W, dec_bond_emb1, dec_bond_emb2,
           dec_W1, dec_b1, dec_W2, dec_b2):
    x = x.astype(_i32)
    src = edge_index[0].astype(_i32)
    dst = edge_index[1].astype(_i32)
    ea = edge_attr.astype(_i32)

    padE = EP - E
    src_p = jnp.concatenate([src, jnp.zeros((padE,), _i32)])
    dst_p = jnp.concatenate([dst, jnp.full((padE,), N, _i32)])
    cval_p = jnp.concatenate([ea[:, 0] * 3 + ea[:, 1],
                              jnp.full((padE,), 15, _i32)])
    perm = jnp.argsort(dst_p, stable=True)
    src_p = src_p[perm]
    dst_p = dst_p[perm]
    cval_p = cval_p[perm]
    dst2 = dst_p.reshape(1280, CH)
    cvals2 = cval_p.reshape(1280, CH)
    src4_l0 = jnp.concatenate(
        [src_p + q * NP for q in range(4)]).reshape(5120, CH)
    src4_ln = jnp.concatenate(
        [src_p + q * N for q in range(4)]).reshape(5120, CH)
    c4 = jnp.concatenate(
        [cval_p + q * 16 for q in range(4)]).reshape(5120, CH)

    ax = x[:, 0] * 3 + x[:, 1]
    ax_p = jnp.concatenate([ax, jnp.zeros((NP - N,), _i32)])
    ax4 = jnp.concatenate([ax_p + q * 360 for q in range(4)])

    atab = (atom_emb1[:, None, :] + atom_emb2[None, :, :]).reshape(360, F)
    atab_sp = _split_cols(atab).reshape(360, 4, FQ)
    atab4 = atab_sp.transpose(1, 0, 2).reshape(4 * 360, FQ)

    bt = bond_emb1[:, :, None, :] + bond_emb2[:, None, :, :]
    btc = jnp.concatenate([bt[:, :3, :, :].reshape(NL, 9, F),
                           jnp.zeros((NL, 7, F), _f32)], axis=1)
    selfr = bt[:, 4, 0, :][:, None, :]

    dbt = dec_bond_emb1[:, None, :] + dec_bond_emb2[None, :, :]
    dbtc = jnp.concatenate([dbt[:3, :, :].reshape(9, F),
                            jnp.zeros((7, F), _f32)], axis=0)
    dselfr = dbt[4, 0, :][None, :]

    btq = _split_cols(btc).reshape(NL, 16, 4, FQ).transpose(0, 2, 1, 3)
    btq = btq.reshape(NL, 64, FQ)
    dbtq = _split_cols(dbtc).reshape(16, 4, FQ).transpose(1, 0, 2)
    dbtq = dbtq.reshape(64, FQ)

    b1r = b1[:, None, :]
    b2r = b2[:, None, :]
    scr = bn_scale[:, None, :]
    bir = bn_bias[:, None, :]
    dW2p = jnp.concatenate(
        [dec_W2, jnp.zeros((HID, OP - ODIM), _f32)], axis=1)
    db1r = dec_b1[None, :]
    db2p = jnp.concatenate([dec_b2, jnp.zeros((OP - ODIM,), _f32)])[None, :]
    pa = jnp.reshape(prelu_a.astype(_f32), (1, 1))
    keep = (1.0 - masked_atom_mask.astype(_f32))[:, None]

    h0_flat, cnt_flat = _sc_init_kernel(atab4, ax4, cvals2, dst2)
    h0 = h0_flat.reshape(4, NP, FQ)
    c2 = cnt_flat.reshape(2, NP, 16)

    agg_l0 = _make_sc_agg(4 * NP)
    agg_ln = _make_sc_agg(4 * N)

    h = h0
    for l in range(NL):
        if l == 0:
            aggf = agg_l0(h.reshape(4 * NP, FQ), btq[l], src4_l0, c4, dst2)
        else:
            aggf = agg_ln(h.reshape(4 * N, FQ), btq[l], src4_ln, c4, dst2)
        aggsc = aggf.reshape(4, NP, FQ)
        mode = "relu" if l < NL - 1 else "dec"
        h = _layer_call(aggsc, h, selfr[l], W1[l], b1r[l], W2[l],
                        b2r[l], scr[l], bir[l], pa, keep, enc2dec_W, mode)

    aggd = agg_ln(h.reshape(4 * N, FQ), dbtq, src4_ln, c4,
                  dst2).reshape(4, NP, FQ)
    out = _dec_call(aggd, h, dselfr, dec_W1, db1r, dW2p, db2p)
    return out[:, :ODIM]
